# baseline (speedup 1.0000x reference)
import functools

import jax
import jax.numpy as jnp
import numpy as np
from jax.experimental import pallas as pl
from jax.experimental.pallas import tpu as pltpu

_BF = 1000


def _cs_kernel(corners_ref, centers_ref, a_ref, be_ref, g_ref,
               wblk_ref, wt_ref, b_ref, out_ref, *, ns: int, oc: int):
    c = corners_ref[0]
    q = jnp.dot(c, wblk_ref[...], preferred_element_type=jnp.float32)
    cen = centers_ref[0]
    qc = jnp.dot(cen, wt_ref[...], preferred_element_type=jnp.float32)
    acc = None
    for s in range(ns):
        n = s % 3
        q0 = q[:, (3 * n + 0) * oc:(3 * n + 1) * oc]
        q1 = q[:, (3 * n + 1) * oc:(3 * n + 2) * oc]
        q2 = q[:, (3 * n + 2) * oc:(3 * n + 3) * oc]
        x = a_ref[:, s:s + 1] * q0 + be_ref[:, s:s + 1] * q1 + g_ref[:, s:s + 1] * q2
        acc = x if acc is None else jnp.maximum(acc, x)
    out_ref[0] = jnp.maximum(acc - qc + b_ref[...], 0.0)


def kernel(ring_n, neighbor_corners, centers, alpha, beta, gamma, W, b):
    del ring_n
    m, f, nn = neighbor_corners.shape[:3]
    ns = alpha.shape[1]
    oc = W.shape[0]
    corners = neighbor_corners.reshape(m, f, nn * 9)
    wt = W.T
    wblk = jnp.kron(jnp.eye(3 * nn, dtype=W.dtype), wt)
    b2 = b.reshape(1, oc)

    bf = _BF
    grid = (f // bf, m)
    out = pl.pallas_call(
        functools.partial(_cs_kernel, ns=ns, oc=oc),
        grid=grid,
        in_specs=[
            pl.BlockSpec((1, bf, nn * 9), lambda fb, mm: (mm, fb, 0)),
            pl.BlockSpec((1, bf, 3), lambda fb, mm: (mm, fb, 0)),
            pl.BlockSpec((bf, ns), lambda fb, mm: (fb, 0)),
            pl.BlockSpec((bf, ns), lambda fb, mm: (fb, 0)),
            pl.BlockSpec((bf, ns), lambda fb, mm: (fb, 0)),
            pl.BlockSpec((nn * 9, 3 * nn * oc), lambda fb, mm: (0, 0)),
            pl.BlockSpec((3, oc), lambda fb, mm: (0, 0)),
            pl.BlockSpec((1, oc), lambda fb, mm: (0, 0)),
        ],
        out_specs=pl.BlockSpec((1, bf, oc), lambda fb, mm: (mm, fb, 0)),
        out_shape=jax.ShapeDtypeStruct((m, f, oc), jnp.float32),
    )(corners, centers, alpha, beta, gamma, wblk, wt, b2)
    return out


# speedup vs baseline: 3.5247x; 3.5247x over previous
import functools

import jax
import jax.numpy as jnp
from jax.experimental import pallas as pl
from jax.experimental.pallas import tpu as pltpu

_BF = 2000


def _cs_kernel(x_ref, bt_ref, gt_ref, wpack_ref, b2_ref, out_ref,
               *, ns: int, oc: int):
    xx = jnp.concatenate([x_ref[0], x_ref[1]], axis=1)
    q = jnp.dot(xx, wpack_ref[...], preferred_element_type=jnp.float32)
    w2 = 2 * oc
    acc = None
    for s in range(ns):
        n = s % 3
        q0 = q[:, (3 * n + 0) * w2:(3 * n + 1) * w2]
        d1 = q[:, (3 * n + 1) * w2:(3 * n + 2) * w2]
        d2 = q[:, (3 * n + 2) * w2:(3 * n + 3) * w2]
        x = q0 + bt_ref[:, s:s + 1] * d1 + gt_ref[:, s:s + 1] * d2
        acc = x if acc is None else jnp.maximum(acc, x)
    qc = q[:, 9 * w2:]
    r = jnp.maximum(acc - qc + b2_ref[...], 0.0)
    out_ref[0] = r[:, :oc]
    out_ref[1] = r[:, oc:]


def kernel(ring_n, neighbor_corners, centers, alpha, beta, gamma, W, b):
    del ring_n, alpha
    m, f, nn = neighbor_corners.shape[:3]
    ns = beta.shape[1]
    oc = W.shape[0]
    nb = 3 * nn + 1
    x30 = jnp.concatenate(
        [neighbor_corners.reshape(m, f, nn * 9), centers], axis=-1)

    tm = jnp.zeros((nb, nb), jnp.float32)
    for n in range(nn):
        tm = tm.at[3 * n + 0, 3 * n + 0].set(1.0)
        tm = tm.at[3 * n + 0, 3 * n + 1].set(-1.0)
        tm = tm.at[3 * n + 0, 3 * n + 2].set(-1.0)
        tm = tm.at[3 * n + 1, 3 * n + 1].set(1.0)
        tm = tm.at[3 * n + 2, 3 * n + 2].set(1.0)
    tm = tm.at[nb - 1, nb - 1].set(1.0)
    wt10 = jnp.kron(tm, W.T)
    wt10r = wt10.reshape(3 * nb, nb, oc)
    z = jnp.zeros_like(wt10r)
    top = jnp.concatenate([wt10r, z], axis=-1).reshape(3 * nb, nb * 2 * oc)
    bot = jnp.concatenate([z, wt10r], axis=-1).reshape(3 * nb, nb * 2 * oc)
    wpack = jnp.concatenate([top, bot], axis=0)
    b2 = jnp.concatenate([b, b]).reshape(1, 2 * oc)

    bf = _BF
    grid = (f // bf, m // 2)
    out = pl.pallas_call(
        functools.partial(_cs_kernel, ns=ns, oc=oc),
        grid=grid,
        in_specs=[
            pl.BlockSpec((2, bf, 3 * nb), lambda fb, mp: (mp, fb, 0)),
            pl.BlockSpec((bf, ns), lambda fb, mp: (fb, 0)),
            pl.BlockSpec((bf, ns), lambda fb, mp: (fb, 0)),
            pl.BlockSpec((6 * nb, nb * 2 * oc), lambda fb, mp: (0, 0)),
            pl.BlockSpec((1, 2 * oc), lambda fb, mp: (0, 0)),
        ],
        out_specs=pl.BlockSpec((2, bf, oc), lambda fb, mp: (mp, fb, 0)),
        out_shape=jax.ShapeDtypeStruct((m, f, oc), jnp.float32),
    )(x30, beta, gamma, wpack, b2)
    return out


# speedup vs baseline: 6.6386x; 1.8835x over previous
import functools

import jax
import jax.numpy as jnp
from jax.experimental import pallas as pl
from jax.experimental.pallas import tpu as pltpu

_BF = 2000
_CH = 128


def _cs_kernel(x_ref, bt_ref, gt_ref, wt_ref, bb_ref, out_ref,
               *, ns: int, oc: int, nn: int):
    xt = x_ref[0].T
    bt_ref = bt_ref[...].T
    gt_ref = gt_ref[...].T
    qt = jnp.dot(wt_ref[...], xt, preferred_element_type=jnp.float32)
    bf = xt.shape[1]
    nk = ns // nn
    for c in range(0, bf, _CH):
        w = min(_CH, bf - c)
        acc = None
        for n in range(nn):
            q0 = qt[(3 * n + 0) * oc:(3 * n + 1) * oc, c:c + w]
            d1 = qt[(3 * n + 1) * oc:(3 * n + 2) * oc, c:c + w]
            d2 = qt[(3 * n + 2) * oc:(3 * n + 3) * oc, c:c + w]
            accn = None
            for k in range(nk):
                s = n + nn * k
                x = bt_ref[s:s + 1, c:c + w] * d1 + gt_ref[s:s + 1, c:c + w] * d2
                accn = x if accn is None else jnp.maximum(accn, x)
            an = accn + q0
            acc = an if acc is None else jnp.maximum(acc, an)
        qc = qt[3 * nn * oc:(3 * nn + 1) * oc, c:c + w]
        r = jnp.maximum(acc - qc + bb_ref[:, :w], 0.0)
        out_ref[0, c:c + w, :] = r.T


def kernel(ring_n, neighbor_corners, centers, alpha, beta, gamma, W, b):
    del ring_n, alpha
    m, f, nn = neighbor_corners.shape[:3]
    ns = beta.shape[1]
    oc = W.shape[0]
    nb = 3 * nn + 1
    x30 = jnp.concatenate(
        [neighbor_corners.reshape(m, f, nn * 9), centers], axis=-1)

    tm = jnp.zeros((nb, nb), jnp.float32)
    for n in range(nn):
        tm = tm.at[3 * n + 0, 3 * n + 0].set(1.0)
        tm = tm.at[3 * n + 0, 3 * n + 1].set(-1.0)
        tm = tm.at[3 * n + 0, 3 * n + 2].set(-1.0)
        tm = tm.at[3 * n + 1, 3 * n + 1].set(1.0)
        tm = tm.at[3 * n + 2, 3 * n + 2].set(1.0)
    tm = tm.at[nb - 1, nb - 1].set(1.0)
    wt = jnp.kron(tm, W.T).T
    bb = jnp.broadcast_to(b[:, None], (oc, _CH))

    bf = _BF
    grid = (f // bf, m)
    out = pl.pallas_call(
        functools.partial(_cs_kernel, ns=ns, oc=oc, nn=nn),
        grid=grid,
        in_specs=[
            pl.BlockSpec((1, bf, 3 * nb), lambda fb, mm: (mm, fb, 0)),
            pl.BlockSpec((bf, ns), lambda fb, mm: (fb, 0)),
            pl.BlockSpec((bf, ns), lambda fb, mm: (fb, 0)),
            pl.BlockSpec((nb * oc, 3 * nb), lambda fb, mm: (0, 0)),
            pl.BlockSpec((oc, _CH), lambda fb, mm: (0, 0)),
        ],
        out_specs=pl.BlockSpec((1, bf, oc), lambda fb, mm: (mm, fb, 0)),
        out_shape=jax.ShapeDtypeStruct((m, f, oc), jnp.float32),
    )(x30, beta, gamma, wt, bb)
    return out
